# Initial kernel scaffold; baseline (speedup 1.0000x reference)
import jax
import jax.numpy as jnp
from jax.experimental import pallas as pl


def kernel(memory, city_coords):
    raise NotImplementedError("write your pallas kernel here")


# speedup vs baseline: 1.1817x; 1.1817x over previous
import functools

import jax
import jax.numpy as jnp
from jax import lax
from jax.experimental import pallas as pl
from jax.experimental.pallas import tpu as pltpu
from jax.experimental.pallas import tpu_sc as plsc

_B = 64
_N = 1024
_K = 4
_PER = _N // _K
_ROUTE = _PER + 2
_NC = 2
_NS = 16
_NW = _NC * _NS
_BPW = _B // _NW
_VECS = _N // 16


def _sort_body(keys_hbm, coords_hbm, out_hbm,
               key_a, val_a, key_b, val_b, coords_v, route_v, hist, off):
  wid = lax.axis_index("s") * _NC + lax.axis_index("c")
  iota = lax.iota(jnp.int32, 16)
  zeros16 = jnp.zeros((16,), jnp.int32)
  ones16 = zeros16 + 1

  for i in range(_BPW):
    b = wid * _BPW + i
    pltpu.sync_copy(keys_hbm.at[b], key_a)
    pltpu.sync_copy(coords_hbm.at[b], coords_v)

    def init_body(j, _):
      val_a[pl.ds(j * 16, 16)] = j * 16 + iota
      return 0
    lax.fori_loop(0, _VECS, init_body, 0)

    for p in range(4):
      src_k, src_v = (key_a, val_a) if p % 2 == 0 else (key_b, val_b)
      dst_k, dst_v = (key_b, val_b) if p % 2 == 0 else (key_a, val_a)
      shift = 8 * p

      for t in range(16):
        hist[pl.ds(t * 16, 16)] = zeros16

      def hist_body(j, _, src_k=src_k, shift=shift):
        kv = src_k[pl.ds(j * 16, 16)]
        dig = (kv >> shift) & 255
        rc, lastm = plsc.scan_count(dig)
        plsc.addupdate_scatter(hist, [dig], rc, mask=lastm)
        return 0
      lax.fori_loop(0, _VECS, hist_body, 0)

      total = jnp.int32(0)
      for t in range(16):
        h = hist[pl.ds(t * 16, 16)]
        off[pl.ds(t * 16, 16)] = plsc.cumsum(h) - h + total
        total = total + jnp.sum(h)

      def perm_body(j, _, src_k=src_k, src_v=src_v,
                    dst_k=dst_k, dst_v=dst_v, shift=shift):
        kv = src_k[pl.ds(j * 16, 16)]
        vv = src_v[pl.ds(j * 16, 16)]
        dig = (kv >> shift) & 255
        rc, lastm = plsc.scan_count(dig)
        pos = plsc.load_gather(off, [dig]) + rc - 1
        plsc.store_scatter(dst_k, [pos], kv)
        plsc.store_scatter(dst_v, [pos], vv)
        plsc.store_scatter(off, [dig], pos + 1, mask=lastm)
        return 0
      lax.fori_loop(0, _VECS, perm_body, 0)

    def gather_body(j, _):
      idx = val_a[pl.ds(j * 16, 16)]
      xoff = idx * 2
      x = plsc.load_gather(coords_v, [xoff])
      y = plsc.load_gather(coords_v, [xoff + 1])
      base = (j // (_PER // 16)) * (_ROUTE * 2) + ((j % (_PER // 16)) * 16 + 1) * 2
      foff = base + iota * 2
      plsc.store_scatter(route_v, [foff], x)
      plsc.store_scatter(route_v, [foff + 1], y)
      return 0
    lax.fori_loop(0, _VECS, gather_body, 0)

    s4 = iota >> 2
    r = iota & 3
    doff = s4 * (_ROUTE * 2) + jnp.where(r < 2, r, (_ROUTE - 1) * 2 - 2 + r)
    plsc.store_scatter(route_v, [doff], jnp.zeros((16,), jnp.float32))

    pltpu.sync_copy(route_v, out_hbm.at[pl.ds(b * _K * _ROUTE * 2, _K * _ROUTE * 2)])


@jax.jit
def kernel(memory, city_coords):
  del memory
  depot = jnp.zeros((2,), dtype=jnp.float32)
  dist = jnp.linalg.norm(city_coords - depot[None, None, :], axis=-1)
  keys = lax.bitcast_convert_type(dist, jnp.int32)

  mesh = plsc.VectorSubcoreMesh(
      core_axis_name="c", subcore_axis_name="s",
      num_cores=_NC, num_subcores=_NS)
  sort_gather = pl.kernel(
      _sort_body,
      out_type=jax.ShapeDtypeStruct((_B * _K * _ROUTE * 2,), jnp.float32),
      mesh=mesh,
      compiler_params=pltpu.CompilerParams(needs_layout_passes=False),
      scratch_types=[
          pltpu.VMEM((_N,), jnp.int32),
          pltpu.VMEM((_N,), jnp.int32),
          pltpu.VMEM((_N,), jnp.int32),
          pltpu.VMEM((_N,), jnp.int32),
          pltpu.VMEM((_N * 2,), jnp.float32),
          pltpu.VMEM((_K * _ROUTE * 2,), jnp.float32),
          pltpu.VMEM((256,), jnp.int32),
          pltpu.VMEM((256,), jnp.int32),
      ],
  )
  routes_flat = sort_gather(keys, city_coords.reshape(_B, _N * 2))
  return routes_flat.reshape(_B * _K, _ROUTE, 2)
